# baseline (speedup 1.0000x reference)
import functools

import jax
import jax.numpy as jnp
from jax import lax
from jax.experimental import pallas as pl
from jax.experimental.pallas import tpu as pltpu
from jax.experimental.pallas import tpu_sc as plsc

_K = 64
_ROWS = 128
_COLS = 32768
_CHUNK = 128
_NCHUNK = _COLS // _CHUNK
_CAP = 128
_NW = 32
_RPW = _ROWS // _NW


def _key(v):
    b = lax.bitcast_convert_type(v, jnp.uint32)
    neg = b >= jnp.uint32(0x80000000)
    return jnp.where(neg, ~b, b | jnp.uint32(0x80000000))


def _chunk_max_block(x_ref, o_ref):
    o_ref[...] = jnp.max(x_ref[...], axis=1, keepdims=True)


def _sc_threshold_body(x2, m, tout, mbuf, cidx, cidx2, gbuf, surv, tvmem, sem):
    wid = lax.axis_index("s") * 2 + lax.axis_index("c")
    lanes = lax.iota(jnp.int32, 16)
    zero16u = jnp.zeros((16,), jnp.uint32)
    zero16i = jnp.zeros((16,), jnp.int32)

    pltpu.sync_copy(m.at[pl.ds(wid * (_RPW * _NCHUNK), _RPW * _NCHUNK)], mbuf)

    tlos = []
    ncs = []
    copies = []
    scope1 = jax.named_scope("sc_phase1")
    scope1.__enter__()
    for j in range(_RPW):
        base = (wid * _RPW + j) * _NCHUNK
        mk = [_key(mbuf[pl.ds(j * _NCHUNK + i * 16, 16)])
              for i in range(_NCHUNK // 16)]

        def tlo_body(it, p, mk=mk):
            bit = (jnp.int32(31) - it).astype(jnp.uint32)
            c = p | (jnp.uint32(1) << bit)
            cnt = zero16i
            for u in mk:
                cnt = cnt + plsc.all_reduce_population_count(u >= c)
            return jnp.where(cnt >= _K, c, p)

        tlo = lax.fori_loop(0, 32, tlo_body, zero16u)
        tlos.append(tlo)

        for i in range(_CAP // 16):
            cidx[pl.ds(j * _CAP + i * 16, 16)] = zero16i
        ncv = zero16i
        for i in range(_NCHUNK // 16):
            msk = mk[i] >= tlo
            mi = msk.astype(jnp.int32)
            pos = ncv + plsc.cumsum(mi) - mi
            plsc.store_scatter(cidx, [pos + j * _CAP], lanes + (base + i * 16),
                               mask=msk & (pos < _CAP))
            plsc.store_scatter(cidx2.at[j], [pos - _CAP], lanes + (base + i * 16),
                               mask=msk & (pos >= _CAP))
            ncv = ncv + plsc.all_reduce_population_count(msk)
        ncs.append(jnp.max(ncv))

    scope1.__exit__(None, None, None)
    gcopy = pltpu.async_copy(x2.at[cidx], gbuf, sem)

    tvec = jnp.zeros((16,), jnp.float32)
    for j in range(_RPW):
        tlo = tlos[j]
        tlo_s = jnp.max(tlo)
        nc = ncs[j]
        if j == 0:
            gcopy.wait()
        scc = jax.named_scope(f"sc_comp{j}")
        scc.__enter__()

        def comp_body(ci, ns, j=j, tlo=tlo):
            for l in range(_CHUNK // 16):
                u = _key(gbuf[j * _CAP + ci, pl.ds(l * 16, 16)])
                msk = u >= tlo
                mi = msk.astype(jnp.int32)
                pos = ns + plsc.cumsum(mi) - mi
                plsc.store_scatter(surv, [pos], plsc.bitcast(u, jnp.int32),
                                   mask=msk)
                ns = ns + plsc.all_reduce_population_count(msk)
            return ns

        nsv = lax.fori_loop(0, jnp.minimum(nc, _CAP), comp_body, zero16i)

        def more(nsv, j=j, nc=nc):
            pltpu.async_copy(x2.at[cidx2.at[j]],
                             gbuf.at[pl.ds(j * _CAP, _CAP)], sem).wait()
            return lax.fori_loop(0, nc - _CAP, comp_body, nsv)

        nsv = lax.cond(nc > _CAP, more, lambda v: v, nsv)
        ns = jnp.max(nsv)
        scc.__exit__(None, None, None)
        scs_ = jax.named_scope(f"sc_sel{j}")
        scs_.__enter__()

        for t in range(4):
            plsc.store_scatter(surv, [ns + t * 16 + lanes], zero16i,
                               mask=lanes >= 0)
        nvp = (ns + 63) // 64

        def sel_body(it, p):
            bit = (jnp.int32(31) - it).astype(jnp.uint32)
            c = p | (jnp.uint32(1) << bit)

            def sure(c=c):
                return c

            def count(c=c, p=p):
                def cnt_body(w, cnt):
                    for l in range(4):
                        u = plsc.bitcast(surv[pl.ds(w * 64 + l * 16, 16)],
                                         jnp.uint32)
                        cnt = cnt + plsc.all_reduce_population_count(u >= c)
                    return cnt

                cnt = lax.fori_loop(0, nvp, cnt_body, zero16i)
                return jnp.where(jnp.max(cnt) >= _K, c, p)

            return lax.cond(c <= tlo_s, sure, count)

        tkey = lax.fori_loop(0, 32, sel_body, jnp.uint32(0))

        tkv = jnp.full((16,), tkey, dtype=jnp.uint32)
        tbits = jnp.where(tkv >= jnp.uint32(0x80000000),
                          tkv & jnp.uint32(0x7FFFFFFF), ~tkv)
        tval = lax.bitcast_convert_type(tbits, jnp.float32)
        tvec = jnp.where(lanes == j, tval, tvec)
        scs_.__exit__(None, None, None)

    tvmem[pl.ds(0, 16)] = tvec
    pltpu.sync_copy(tvmem.at[pl.ds(0, 16)], tout.at[wid])


_sc_threshold = functools.partial(
    pl.kernel,
    out_type=jax.ShapeDtypeStruct((_NW, 16), jnp.float32),
    mesh=plsc.VectorSubcoreMesh(core_axis_name="c", subcore_axis_name="s"),
    compiler_params=pltpu.CompilerParams(
        needs_layout_passes=False, use_tc_tiling_on_sc=False),
    scratch_types=[
        pltpu.VMEM((_RPW * _NCHUNK,), jnp.float32),
        pltpu.VMEM((_RPW * _CAP,), jnp.int32),
        pltpu.VMEM((_RPW, _CAP), jnp.int32),
        pltpu.VMEM((_RPW * _CAP, _CHUNK), jnp.float32),
        pltpu.VMEM((_COLS + 80,), jnp.int32),
        pltpu.VMEM((16,), jnp.float32),
        pltpu.SemaphoreType.DMA,
    ],
)(_sc_threshold_body)


def _mask_block(x_ref, t_ref, o_ref):
    xb = x_ref[...]
    o_ref[...] = jnp.where(xb >= t_ref[...], xb, jnp.zeros_like(xb))


@jax.jit
def kernel(x):
    x2 = x.reshape(_ROWS * _NCHUNK, _CHUNK)

    chunk_max = pl.pallas_call(
        _chunk_max_block,
        grid=(16,),
        in_specs=[pl.BlockSpec((_ROWS * _NCHUNK // 16, _CHUNK), lambda i: (i, 0))],
        out_specs=pl.BlockSpec((_ROWS * _NCHUNK // 16, 1), lambda i: (i, 0)),
        out_shape=jax.ShapeDtypeStruct((_ROWS * _NCHUNK, 1), jnp.float32),
    )(x2)

    tout = _sc_threshold(x2, chunk_max.reshape(_ROWS * _NCHUNK))
    thresholds = tout[:, :_RPW].reshape(_ROWS, 1)

    return pl.pallas_call(
        _mask_block,
        grid=(16,),
        in_specs=[
            pl.BlockSpec((_ROWS // 16, _COLS), lambda i: (i, 0)),
            pl.BlockSpec((_ROWS // 16, 1), lambda i: (i, 0)),
        ],
        out_specs=pl.BlockSpec((_ROWS // 16, _COLS), lambda i: (i, 0)),
        out_shape=jax.ShapeDtypeStruct(x.shape, x.dtype),
    )(x, thresholds)


# speedup vs baseline: 1.2655x; 1.2655x over previous
import jax
import jax.numpy as jnp
from jax.experimental import pallas as pl

_K = 64
_ROWS_PER_BLOCK = 8


def _sortable_key(x):
    b = jax.lax.bitcast_convert_type(x, jnp.uint32)
    neg = b >= jnp.uint32(0x80000000)
    return jnp.where(neg, ~b, b | jnp.uint32(0x80000000))


def _topk_mask_block(x_ref, o_ref):
    xb = x_ref[...]
    u = _sortable_key(xb)
    rows, cols = xb.shape

    mu = _sortable_key(jnp.max(xb.reshape(rows, cols // 128, 128), axis=-1))

    def tlo_body(i, p):
        bit = jnp.uint32(31) - i.astype(jnp.uint32)
        c = p | (jnp.uint32(1) << bit)
        cnt = jnp.sum((mu >= c).astype(jnp.int32), axis=1, keepdims=True)
        return jnp.where(cnt >= _K, c, p)

    tlo = jax.lax.fori_loop(
        0, 32, tlo_body, jnp.zeros((rows, 1), dtype=jnp.uint32))

    def cond(state):
        i, _, cnt_p = state
        return jnp.logical_and(i < 32, jnp.logical_not(jnp.all(cnt_p == _K)))

    def body(state):
        i, p, cnt_p = state
        bit = jnp.uint32(31) - i.astype(jnp.uint32)
        c = p | (jnp.uint32(1) << bit)
        acc = c <= tlo

        def sure(c=c, cnt_p=cnt_p):
            return c, cnt_p

        def count(c=c, p=p, acc=acc, cnt_p=cnt_p):
            cnt = jnp.sum((u >= c).astype(jnp.int32), axis=1, keepdims=True)
            take = jnp.logical_or(acc, cnt >= _K)
            return jnp.where(take, c, p), jnp.where(take, cnt, cnt_p)

        p, cnt_p = jax.lax.cond(jnp.all(acc), sure, count)
        return i + 1, p, cnt_p

    p0 = jnp.zeros((rows, 1), dtype=jnp.uint32)
    c0 = jnp.full((rows, 1), cols, dtype=jnp.int32)
    _, p, _ = jax.lax.while_loop(cond, body, (0, p0, c0))
    o_ref[...] = jnp.where(u >= p, xb, jnp.zeros_like(xb))


@jax.jit
def kernel(x):
    n_rows, n_cols = x.shape
    grid = (n_rows // _ROWS_PER_BLOCK,)
    return pl.pallas_call(
        _topk_mask_block,
        grid=grid,
        in_specs=[pl.BlockSpec((_ROWS_PER_BLOCK, n_cols), lambda i: (i, 0))],
        out_specs=pl.BlockSpec((_ROWS_PER_BLOCK, n_cols), lambda i: (i, 0)),
        out_shape=jax.ShapeDtypeStruct(x.shape, x.dtype),
    )(x)


# speedup vs baseline: 2.4641x; 1.9472x over previous
import jax
import jax.numpy as jnp
from jax.experimental import pallas as pl

_K = 64
_ROWS_PER_BLOCK = 8


def _sortable_key(x):
    b = jax.lax.bitcast_convert_type(x, jnp.uint32)
    neg = b >= jnp.uint32(0x80000000)
    return jnp.where(neg, ~b, b | jnp.uint32(0x80000000))


def _topk_mask_block(x_ref, o_ref):
    xb = x_ref[...]
    u = _sortable_key(xb)
    rows, cols = xb.shape

    mf = xb
    while mf.shape[1] > 128:
        h = mf.shape[1] // 2
        mf = jnp.maximum(mf[:, :h], mf[:, h:])
    mu = _sortable_key(mf)

    def tlo_body(i, p):
        bit = jnp.uint32(31) - i.astype(jnp.uint32)
        c = p | (jnp.uint32(1) << bit)
        cnt = jnp.sum((mu >= c).astype(jnp.int32), axis=1, keepdims=True)
        return jnp.where(cnt >= _K, c, p)

    tlo = jax.lax.fori_loop(
        0, 32, tlo_body, jnp.zeros((rows, 1), dtype=jnp.uint32))

    def cond(state):
        i, _, cnt_p = state
        return jnp.logical_and(i < 32, jnp.logical_not(jnp.all(cnt_p == _K)))

    def body(state):
        i, p, cnt_p = state
        bit = jnp.uint32(31) - i.astype(jnp.uint32)
        c = p | (jnp.uint32(1) << bit)
        acc = c <= tlo

        def sure(c=c, cnt_p=cnt_p):
            return c, cnt_p

        def count(c=c, p=p, acc=acc, cnt_p=cnt_p):
            cnt = jnp.sum((u >= c).astype(jnp.int32), axis=1, keepdims=True)
            take = jnp.logical_or(acc, cnt >= _K)
            return jnp.where(take, c, p), jnp.where(take, cnt, cnt_p)

        p, cnt_p = jax.lax.cond(jnp.all(acc), sure, count)
        return i + 1, p, cnt_p

    p0 = jnp.zeros((rows, 1), dtype=jnp.uint32)
    c0 = jnp.full((rows, 1), cols, dtype=jnp.int32)
    _, p, _ = jax.lax.while_loop(cond, body, (0, p0, c0))
    o_ref[...] = jnp.where(u >= p, xb, jnp.zeros_like(xb))


@jax.jit
def kernel(x):
    n_rows, n_cols = x.shape
    grid = (n_rows // _ROWS_PER_BLOCK,)
    return pl.pallas_call(
        _topk_mask_block,
        grid=grid,
        in_specs=[pl.BlockSpec((_ROWS_PER_BLOCK, n_cols), lambda i: (i, 0))],
        out_specs=pl.BlockSpec((_ROWS_PER_BLOCK, n_cols), lambda i: (i, 0)),
        out_shape=jax.ShapeDtypeStruct(x.shape, x.dtype),
    )(x)


# speedup vs baseline: 3.2702x; 1.3271x over previous
import functools

import jax
import jax.numpy as jnp
from jax.experimental import pallas as pl

_K = 64
_ROWS_PER_BLOCK = 8


def _sortable_key(x):
    b = jax.lax.bitcast_convert_type(x, jnp.uint32)
    neg = b >= jnp.uint32(0x80000000)
    return jnp.where(neg, ~b, b | jnp.uint32(0x80000000))


def _topk_mask_block(x_ref, o_ref):
    xb = x_ref[...]
    u = _sortable_key(xb)

    def cond(state):
        i, _, cnt_p = state
        return jnp.logical_and(i < 32, jnp.logical_not(jnp.all(cnt_p == _K)))

    def body(state):
        i, p, cnt_p = state
        bit = jnp.uint32(31) - i.astype(jnp.uint32)
        c = p | (jnp.uint32(1) << bit)
        cnt = jnp.sum((u >= c).astype(jnp.int32), axis=1, keepdims=True)
        take = cnt >= _K
        return i + 1, jnp.where(take, c, p), jnp.where(take, cnt, cnt_p)

    p0 = jnp.zeros((xb.shape[0], 1), dtype=jnp.uint32)
    c0 = jnp.full((xb.shape[0], 1), xb.shape[1], dtype=jnp.int32)
    _, p, _ = jax.lax.while_loop(cond, body, (0, p0, c0))
    o_ref[...] = jnp.where(u >= p, xb, jnp.zeros_like(xb))


@jax.jit
def kernel(x):
    n_rows, n_cols = x.shape
    grid = (n_rows // _ROWS_PER_BLOCK,)
    return pl.pallas_call(
        _topk_mask_block,
        grid=grid,
        in_specs=[pl.BlockSpec((_ROWS_PER_BLOCK, n_cols), lambda i: (i, 0))],
        out_specs=pl.BlockSpec((_ROWS_PER_BLOCK, n_cols), lambda i: (i, 0)),
        out_shape=jax.ShapeDtypeStruct(x.shape, x.dtype),
    )(x)


# speedup vs baseline: 5.5057x; 1.6836x over previous
import functools

import jax
import jax.numpy as jnp
from jax.experimental import pallas as pl

_K = 64
_ROWS_PER_BLOCK = 16


def _sortable_key(x):
    b = jax.lax.bitcast_convert_type(x, jnp.uint32)
    neg = b >= jnp.uint32(0x80000000)
    return jnp.where(neg, ~b, b | jnp.uint32(0x80000000))


def _topk_mask_block(x_ref, o_ref):
    xb = x_ref[...]
    u = _sortable_key(xb)

    def cond(state):
        i, _, cnt_p = state
        return jnp.logical_and(i < 32, jnp.logical_not(jnp.all(cnt_p == _K)))

    def body(state):
        i, p, cnt_p = state
        bit = jnp.uint32(31) - i.astype(jnp.uint32)
        c = p | (jnp.uint32(1) << bit)
        cnt = jnp.sum((u >= c).astype(jnp.int32), axis=1, keepdims=True)
        take = cnt >= _K
        return i + 1, jnp.where(take, c, p), jnp.where(take, cnt, cnt_p)

    p0 = jnp.zeros((xb.shape[0], 1), dtype=jnp.uint32)
    c0 = jnp.full((xb.shape[0], 1), xb.shape[1], dtype=jnp.int32)
    _, p, _ = jax.lax.while_loop(cond, body, (0, p0, c0))
    o_ref[...] = jnp.where(u >= p, xb, jnp.zeros_like(xb))


@jax.jit
def kernel(x):
    n_rows, n_cols = x.shape
    grid = (n_rows // _ROWS_PER_BLOCK,)
    return pl.pallas_call(
        _topk_mask_block,
        grid=grid,
        in_specs=[pl.BlockSpec((_ROWS_PER_BLOCK, n_cols), lambda i: (i, 0))],
        out_specs=pl.BlockSpec((_ROWS_PER_BLOCK, n_cols), lambda i: (i, 0)),
        out_shape=jax.ShapeDtypeStruct(x.shape, x.dtype),
    )(x)
